# baseline (speedup 1.0000x reference)
import functools

import jax
import jax.numpy as jnp
from jax import lax
from jax.experimental import pallas as pl
from jax.experimental.pallas import tpu as pltpu
from jax.experimental.pallas import tpu_sc as plsc

_BATCH = 32
_OBS_DIM = 12
_VOCAB = 10
_EMBED = 16
_OUT = 128
_NGRP = _OBS_DIM // 3
_GRP_ROWS = _VOCAB ** 3
_TAB_ROWS = _NGRP * _GRP_ROWS
_NTILES = 32
_ROWS_PER_TILE = _OBS_DIM
_IDX_PER_TILE = _ROWS_PER_TILE * _NGRP


def _tc_table_body(obs_ref, emb_ref, w_ref, b_ref, qt_ref, idx_ref):
    emb = emb_ref[...]
    for g in range(_NGRP):
        j1, j2, j3 = g, g + _NGRP, g + 2 * _NGRP
        q = []
        for j in (j1, j2, j3):
            q.append(jax.lax.dot_general(
                emb, w_ref[_EMBED * j:_EMBED * (j + 1), :],
                (((1,), (0,)), ((), ())), preferred_element_type=jnp.float32))
        qa = jnp.broadcast_to(q[0][:, None, None, :],
                              (_VOCAB, _VOCAB, _VOCAB, _OUT))
        qb = jnp.broadcast_to(q[1][None, :, None, :],
                              (_VOCAB, _VOCAB, _VOCAB, _OUT))
        qc = jnp.broadcast_to(q[2][None, None, :, :],
                              (_VOCAB, _VOCAB, _VOCAB, _OUT))
        blk = (qa + qb + qc).reshape(_GRP_ROWS, _OUT)
        if g == 0:
            blk = blk + b_ref[...]
        qt_ref[_GRP_ROWS * g:_GRP_ROWS * (g + 1), :] = blk

    obs3 = obs_ref[...]
    got = lax.broadcasted_iota(jnp.int32, (_BATCH, _OBS_DIM, _NGRP), 2)
    tidx = (obs3[:, :, 0:_NGRP] * 100 + obs3[:, :, _NGRP:2 * _NGRP] * 10
            + obs3[:, :, 2 * _NGRP:3 * _NGRP]
            + got * _GRP_ROWS)
    flat = tidx.reshape(_BATCH, 1, _IDX_PER_TILE)
    idx_ref[:, :, 0:_IDX_PER_TILE] = flat


def _build_table(obs, emb_table, W, b):
    return pl.pallas_call(
        _tc_table_body,
        out_shape=[
            jax.ShapeDtypeStruct((_TAB_ROWS, _OUT), jnp.float32),
            jax.ShapeDtypeStruct((_NTILES, 1, 128), jnp.int32),
        ],
    )(obs, emb_table, W, b.reshape(1, _OUT))


_sc_mesh = plsc.VectorSubcoreMesh(core_axis_name="c", subcore_axis_name="s")


@functools.partial(
    pl.kernel,
    out_type=jax.ShapeDtypeStruct((_BATCH, _OBS_DIM, _OUT), jnp.float32),
    mesh=_sc_mesh,
    scratch_types=[
        pltpu.VMEM((1, 128), jnp.int32),
        pltpu.VMEM((_IDX_PER_TILE, _OUT), jnp.float32),
        pltpu.VMEM((_ROWS_PER_TILE, _OUT), jnp.float32),
        pltpu.SemaphoreType.DMA,
        pltpu.SemaphoreType.DMA,
    ],
)
def _sc_gather_sum(qt_hbm, idx_hbm, out_hbm, idx_v, rows_v, out_v, sem_a,
                   sem_b):
    wid = lax.axis_index("s") * 2 + lax.axis_index("c")
    half = _IDX_PER_TILE // 2
    pltpu.sync_copy(idx_hbm.at[wid], idx_v)
    cp_a = pltpu.async_copy(
        qt_hbm.at[idx_v.at[0, pl.ds(0, half)]],
        rows_v.at[pl.ds(0, half)], sem_a)
    cp_b = pltpu.async_copy(
        qt_hbm.at[idx_v.at[0, pl.ds(half, half)]],
        rows_v.at[pl.ds(half, half)], sem_b)

    def _reduce_rows(r0, r1):
        for r in range(r0, r1):
            for c in range(_OUT // 16):
                s = pl.ds(16 * c, 16)
                acc = rows_v[_NGRP * r, s]
                for t in range(1, _NGRP):
                    acc = acc + rows_v[_NGRP * r + t, s]
                out_v[r, s] = acc

    cp_a.wait()
    _reduce_rows(0, _ROWS_PER_TILE // 2)
    cp_b.wait()
    _reduce_rows(_ROWS_PER_TILE // 2, _ROWS_PER_TILE)
    pltpu.sync_copy(out_v, out_hbm.at[wid])


def kernel(obs, emb_table, W, b):
    qt, idx5 = _build_table(obs.astype(jnp.int32), emb_table, W, b)
    return _sc_gather_sum(qt, idx5)


# speedup vs baseline: 1.0357x; 1.0357x over previous
import functools

import jax
import jax.numpy as jnp
from jax import lax
from jax.experimental import pallas as pl
from jax.experimental.pallas import tpu as pltpu
from jax.experimental.pallas import tpu_sc as plsc

_BATCH = 32
_OBS_DIM = 12
_VOCAB = 10
_EMBED = 16
_OUT = 128
_NPAIR = _OBS_DIM // 2
_PAIR_ROWS = _VOCAB * _VOCAB
_TAB_ROWS = _NPAIR * _PAIR_ROWS
_NTILES = 32
_ROWS_PER_TILE = _OBS_DIM
_IDX_PER_TILE = _ROWS_PER_TILE * _NPAIR


def _tc_table_body(obs_ref, emb_ref, w_ref, b_ref, qp_ref, idx_ref):
    emb = emb_ref[...]
    for p in range(_NPAIR):
        j1, j2 = p, p + _NPAIR
        q1 = jax.lax.dot_general(
            emb, w_ref[_EMBED * j1:_EMBED * (j1 + 1), :],
            (((1,), (0,)), ((), ())), preferred_element_type=jnp.float32)
        q2 = jax.lax.dot_general(
            emb, w_ref[_EMBED * j2:_EMBED * (j2 + 1), :],
            (((1,), (0,)), ((), ())), preferred_element_type=jnp.float32)
        q1r = jnp.broadcast_to(q1[:, None, :], (_VOCAB, _VOCAB, _OUT))
        q2t = jnp.broadcast_to(q2[None, :, :], (_VOCAB, _VOCAB, _OUT))
        blk = (q1r + q2t).reshape(_PAIR_ROWS, _OUT)
        if p == 0:
            blk = blk + b_ref[...]
        qp_ref[_PAIR_ROWS * p:_PAIR_ROWS * (p + 1), :] = blk

    obs3 = obs_ref[...]
    jot = lax.broadcasted_iota(jnp.int32, (_BATCH, _OBS_DIM, _NPAIR), 2)
    pidx = (obs3[:, :, 0:_NPAIR] * _VOCAB + obs3[:, :, _NPAIR:_OBS_DIM]
            + jot * _PAIR_ROWS)
    flat = pidx.reshape(_BATCH, 1, _IDX_PER_TILE)
    idx_ref[:, :, 0:_IDX_PER_TILE] = flat


def _build_table(obs, emb_table, W, b):
    return pl.pallas_call(
        _tc_table_body,
        out_shape=[
            jax.ShapeDtypeStruct((_TAB_ROWS, _OUT), jnp.float32),
            jax.ShapeDtypeStruct((_NTILES, 1, 128), jnp.int32),
        ],
    )(obs, emb_table, W, b.reshape(1, _OUT))


_sc_mesh = plsc.VectorSubcoreMesh(core_axis_name="c", subcore_axis_name="s")


@functools.partial(
    pl.kernel,
    out_type=jax.ShapeDtypeStruct((_BATCH, _OBS_DIM, _OUT), jnp.float32),
    mesh=_sc_mesh,
    scratch_types=[
        pltpu.VMEM((1, 128), jnp.int32),
        pltpu.VMEM((_IDX_PER_TILE, _OUT), jnp.float32),
        pltpu.VMEM((_ROWS_PER_TILE, _OUT), jnp.float32),
        pltpu.SemaphoreType.DMA,
        pltpu.SemaphoreType.DMA,
    ],
)
def _sc_gather_sum(qp_hbm, idx_hbm, out_hbm, idx_v, rows_v, out_v, sem_a,
                   sem_b):
    wid = lax.axis_index("s") * 2 + lax.axis_index("c")
    half = _IDX_PER_TILE // 2
    pltpu.sync_copy(idx_hbm.at[wid], idx_v)
    cp_a = pltpu.async_copy(
        qp_hbm.at[idx_v.at[0, pl.ds(0, half)]],
        rows_v.at[pl.ds(0, half)], sem_a)
    cp_b = pltpu.async_copy(
        qp_hbm.at[idx_v.at[0, pl.ds(half, half)]],
        rows_v.at[pl.ds(half, half)], sem_b)

    def _reduce_rows(r0, r1):
        for r in range(r0, r1):
            for c in range(_OUT // 16):
                s = pl.ds(16 * c, 16)
                acc = rows_v[_NPAIR * r, s]
                for t in range(1, _NPAIR):
                    acc = acc + rows_v[_NPAIR * r + t, s]
                out_v[r, s] = acc

    cp_a.wait()
    _reduce_rows(0, _ROWS_PER_TILE // 2)
    cp_b.wait()
    _reduce_rows(_ROWS_PER_TILE // 2, _ROWS_PER_TILE)
    pltpu.sync_copy(out_v, out_hbm.at[wid])


def kernel(obs, emb_table, W, b):
    qp, idx5 = _build_table(obs.astype(jnp.int32), emb_table, W, b)
    return _sc_gather_sum(qp, idx5)
